# baseline (speedup 1.0000x reference)
import jax
import jax.numpy as jnp
from jax.experimental import pallas as pl

_C_BLK = 32


def _add_kernel(x_ref, emb_ref, o_ref):
    j = pl.program_id(1)
    fe = emb_ref[...].T
    fe_q01 = jnp.where(j == 0, fe[0 * _C_BLK:1 * _C_BLK], fe[1 * _C_BLK:2 * _C_BLK])
    fe_q23 = jnp.where(j == 2, fe[2 * _C_BLK:3 * _C_BLK], fe[3 * _C_BLK:4 * _C_BLK])
    fe_half = jnp.where(j < 2, fe_q01, fe_q23)
    o_ref[...] = x_ref[...] + fe_half[None, :, :, None]


def kernel(x, emb_table):
    b, c, f, t = x.shape
    grid = (b, c // _C_BLK)
    return pl.pallas_call(
        _add_kernel,
        grid=grid,
        in_specs=[
            pl.BlockSpec((1, _C_BLK, f, t), lambda i, j: (i, j, 0, 0)),
            pl.BlockSpec((f, c), lambda i, j: (0, 0)),
        ],
        out_specs=pl.BlockSpec((1, _C_BLK, f, t), lambda i, j: (i, j, 0, 0)),
        out_shape=jax.ShapeDtypeStruct(x.shape, x.dtype),
    )(x, emb_table)
